# Initial kernel scaffold; baseline (speedup 1.0000x reference)
import jax
import jax.numpy as jnp
from jax.experimental import pallas as pl


def kernel(global_emb, local_emb, global_scores, local_scores, candidate_indices, W1, b1, W2, b2):
    raise NotImplementedError("write your pallas kernel here")


# speedup vs baseline: 1.5784x; 1.5784x over previous
import functools

import jax
import jax.numpy as jnp
from jax import lax
from jax.experimental import pallas as pl
from jax.experimental.pallas import tpu as pltpu
from jax.experimental.pallas import tpu_sc as plsc

NUM_NODES = 1000000
NUM_CAND = 65536
GLOBAL_DIM = 128
LOCAL_DIM = 128
HID = 32

NC = 2
NS = 16
NW = NC * NS

CHUNK = 31256
CHUNK_LAST = NUM_NODES - (NW - 1) * CHUNK

CAND_CH = 4096
N_CH = NUM_CAND // CAND_CH

BR = 1024
GRID = NUM_CAND // BR


def _ffn_body(g_ref, l_ref, w1a_ref, w1b_ref, b1_ref, w2_ref, b2_ref, o_ref):
    dn = (((0,), (1,)), ((), ()))
    ht = lax.dot_general(w1a_ref[...], g_ref[...], dn,
                         preferred_element_type=jnp.float32)
    ht = ht + lax.dot_general(w1b_ref[...], l_ref[...], dn,
                              preferred_element_type=jnp.float32)
    ht = jnp.maximum(ht + b1_ref[...], 0.0)
    st = lax.dot_general(w2_ref[...], ht, (((0,), (0,)), ((), ())),
                         preferred_element_type=jnp.float32)
    o_ref[...] = jax.nn.sigmoid(st + b2_ref[...]).reshape(1, 1, BR)


def _compute_sigma(global_emb, local_emb, W1, b1, W2, b2):
    w1a = W1[:GLOBAL_DIM]
    w1b = W1[GLOBAL_DIM:]
    b1c = b1.reshape(HID, 1)
    b2c = b2.reshape(1, 1)
    out = pl.pallas_call(
        _ffn_body,
        grid=(GRID,),
        in_specs=[
            pl.BlockSpec((BR, GLOBAL_DIM), lambda i: (i, 0)),
            pl.BlockSpec((BR, LOCAL_DIM), lambda i: (i, 0)),
            pl.BlockSpec((GLOBAL_DIM, HID), lambda i: (0, 0)),
            pl.BlockSpec((LOCAL_DIM, HID), lambda i: (0, 0)),
            pl.BlockSpec((HID, 1), lambda i: (0, 0)),
            pl.BlockSpec((HID, 1), lambda i: (0, 0)),
            pl.BlockSpec((1, 1), lambda i: (0, 0)),
        ],
        out_specs=pl.BlockSpec((1, 1, BR), lambda i: (i, 0, 0)),
        out_shape=jax.ShapeDtypeStruct((GRID, 1, BR), jnp.float32),
    )(global_emb, local_emb, w1a, w1b, b1c, W2, b2c)
    return out.reshape(NUM_CAND)


def _sc_body(gs_hbm, idx_hbm, sig_hbm, loc_hbm, out_hbm,
             vals_src, vals_dst, idxb, sigb, locb):
    cid = lax.axis_index("c")
    sid = lax.axis_index("s")
    wid = sid * NC + cid
    base = wid * CHUNK
    is_last = wid == NW - 1
    cw = jnp.where(is_last, CHUNK_LAST, CHUNK).astype(jnp.int32)

    @pl.when(jnp.logical_not(is_last))
    def _():
        pltpu.sync_copy(gs_hbm.at[pl.ds(base, CHUNK)], vals_src)
        pltpu.sync_copy(gs_hbm.at[pl.ds(base, CHUNK)], vals_dst)

    @pl.when(is_last)
    def _():
        pltpu.sync_copy(gs_hbm.at[pl.ds(base, CHUNK_LAST)],
                        vals_src.at[pl.ds(0, CHUNK_LAST)])
        pltpu.sync_copy(gs_hbm.at[pl.ds(base, CHUNK_LAST)],
                        vals_dst.at[pl.ds(0, CHUNK_LAST)])

    def chunk_body(c, carry):
        cbase = c * CAND_CH
        pltpu.sync_copy(idx_hbm.at[pl.ds(cbase, CAND_CH)], idxb)
        pltpu.sync_copy(sig_hbm.at[pl.ds(cbase, CAND_CH)], sigb)
        pltpu.sync_copy(loc_hbm.at[pl.ds(cbase, CAND_CH)], locb)

        def vec_body(i, carry2):
            off = i * 16
            iv = idxb[pl.ds(off, 16)]
            sv = sigb[pl.ds(off, 16)]
            lv = locb[pl.ds(off, 16)]
            rel = iv - base
            m = (rel >= 0) & (rel < cw)
            relc = jnp.minimum(jnp.maximum(rel, 0), CHUNK - 1)
            g = plsc.load_gather(vals_src, [relc], mask=m)
            f = sv * g + (1.0 - sv) * lv
            plsc.store_scatter(vals_dst, [relc], f, mask=m)
            return carry2

        return lax.fori_loop(0, CAND_CH // 16, vec_body, carry)

    lax.fori_loop(0, N_CH, chunk_body, 0)

    @pl.when(jnp.logical_not(is_last))
    def _():
        pltpu.sync_copy(vals_dst, out_hbm.at[pl.ds(base, CHUNK)])

    @pl.when(is_last)
    def _():
        pltpu.sync_copy(vals_dst.at[pl.ds(0, CHUNK_LAST)],
                        out_hbm.at[pl.ds(base, CHUNK_LAST)])


_sc_scatter = functools.partial(
    pl.kernel,
    out_type=jax.ShapeDtypeStruct((NUM_NODES,), jnp.float32),
    mesh=plsc.VectorSubcoreMesh(core_axis_name="c", subcore_axis_name="s",
                                num_cores=NC, num_subcores=NS),
    scratch_types=[
        pltpu.VMEM((CHUNK,), jnp.float32),
        pltpu.VMEM((CHUNK,), jnp.float32),
        pltpu.VMEM((CAND_CH,), jnp.int32),
        pltpu.VMEM((CAND_CH,), jnp.float32),
        pltpu.VMEM((CAND_CH,), jnp.float32),
    ],
    compiler_params=pltpu.CompilerParams(needs_layout_passes=False),
)(_sc_body)


def kernel(global_emb, local_emb, global_scores, local_scores,
           candidate_indices, W1, b1, W2, b2):
    sigma = _compute_sigma(global_emb, local_emb, W1, b1, W2, b2)
    idx = candidate_indices.astype(jnp.int32)
    fused = _sc_scatter(global_scores, idx, sigma, local_scores)
    return (fused, sigma)


# speedup vs baseline: 2.0171x; 1.2779x over previous
import functools

import jax
import jax.numpy as jnp
from jax import lax
from jax.experimental import pallas as pl
from jax.experimental.pallas import tpu as pltpu
from jax.experimental.pallas import tpu_sc as plsc

NUM_NODES = 1000000
NUM_CAND = 65536
GLOBAL_DIM = 128
LOCAL_DIM = 128
HID = 32

NC = 2
NS = 16
NW = NC * NS

CHUNK = 31256
CHUNK_LAST = NUM_NODES - (NW - 1) * CHUNK

CAND_CH = 4096
N_CH = NUM_CAND // CAND_CH

BR = 1024
GRID = NUM_CAND // BR


def _ffn_body(g_ref, l_ref, w1a_ref, w1b_ref, b1_ref, w2_ref, b2_ref, o_ref):
    dn = (((0,), (1,)), ((), ()))
    ht = lax.dot_general(w1a_ref[...], g_ref[...], dn,
                         preferred_element_type=jnp.float32)
    ht = ht + lax.dot_general(w1b_ref[...], l_ref[...], dn,
                              preferred_element_type=jnp.float32)
    ht = jnp.maximum(ht + b1_ref[...], 0.0)
    st = lax.dot_general(w2_ref[...], ht, (((0,), (0,)), ((), ())),
                         preferred_element_type=jnp.float32)
    o_ref[...] = jax.nn.sigmoid(st + b2_ref[...]).reshape(1, 1, BR)


def _compute_sigma(global_emb, local_emb, W1, b1, W2, b2):
    w1a = W1[:GLOBAL_DIM]
    w1b = W1[GLOBAL_DIM:]
    b1c = b1.reshape(HID, 1)
    b2c = b2.reshape(1, 1)
    out = pl.pallas_call(
        _ffn_body,
        grid=(GRID,),
        in_specs=[
            pl.BlockSpec((BR, GLOBAL_DIM), lambda i: (i, 0)),
            pl.BlockSpec((BR, LOCAL_DIM), lambda i: (i, 0)),
            pl.BlockSpec((GLOBAL_DIM, HID), lambda i: (0, 0)),
            pl.BlockSpec((LOCAL_DIM, HID), lambda i: (0, 0)),
            pl.BlockSpec((HID, 1), lambda i: (0, 0)),
            pl.BlockSpec((HID, 1), lambda i: (0, 0)),
            pl.BlockSpec((1, 1), lambda i: (0, 0)),
        ],
        out_specs=pl.BlockSpec((1, 1, BR), lambda i: (i, 0, 0)),
        out_shape=jax.ShapeDtypeStruct((GRID, 1, BR), jnp.float32),
    )(global_emb, local_emb, w1a, w1b, b1c, W2, b2c)
    return out.reshape(NUM_CAND)


CPT = NUM_CAND // NS
GROWS = CPT // 128
ROWS_CH = CAND_CH // 128


def _sc_body(gs_hbm, idx2_hbm, sig_hbm, loc_hbm, out_hbm,
             fused_sh, vals_dst, idxg, sigb, locb, gbuf, fusedb,
             idxsb, fuseds, gsem):
    cid = lax.axis_index("c")
    sid = lax.axis_index("s")
    wid = sid * NC + cid
    base = wid * CHUNK
    is_last = wid == NW - 1
    cw = jnp.where(is_last, CHUNK_LAST, CHUNK).astype(jnp.int32)

    abase = sid * CPT
    rbase = sid * GROWS
    pltpu.sync_copy(idx2_hbm.at[pl.ds(rbase, GROWS)], idxg)
    pltpu.sync_copy(sig_hbm.at[pl.ds(abase, CPT)], sigb)
    pltpu.sync_copy(loc_hbm.at[pl.ds(abase, CPT)], locb)

    @pl.loop(0, GROWS)
    def _(j):
        pltpu.async_copy(gs_hbm.at[idxg.at[j]], gbuf.at[j], gsem)

    @pl.loop(0, GROWS)
    def _(j):
        pltpu.make_async_copy(gs_hbm.at[idxg.at[j]], gbuf.at[j], gsem).wait()

    def fuse_row(j, carry):
        def fuse_vec(k, carry2):
            off = k * 16
            sv = sigb[pl.ds(j * 128 + off, 16)]
            lv = locb[pl.ds(j * 128 + off, 16)]
            gv = gbuf[j, pl.ds(off, 16)]
            fusedb[pl.ds(j * 128 + off, 16)] = sv * gv + (1.0 - sv) * lv
            return carry2
        return lax.fori_loop(0, 8, fuse_vec, carry)

    lax.fori_loop(0, GROWS, fuse_row, 0)
    pltpu.sync_copy(fusedb, fused_sh.at[pl.ds(abase, CPT)])

    @pl.when(jnp.logical_not(is_last))
    def _():
        pltpu.sync_copy(gs_hbm.at[pl.ds(base, CHUNK)], vals_dst)

    @pl.when(is_last)
    def _():
        pltpu.sync_copy(gs_hbm.at[pl.ds(base, CHUNK_LAST)],
                        vals_dst.at[pl.ds(0, CHUNK_LAST)])

    plsc.subcore_barrier()

    def chunk_body(c, carry):
        slot = lax.rem(c, 2)
        cbase = c * CAND_CH
        pltpu.sync_copy(idx2_hbm.at[pl.ds(c * ROWS_CH, ROWS_CH)],
                        idxsb.at[slot])
        pltpu.sync_copy(fused_sh.at[pl.ds(cbase, CAND_CH)],
                        fuseds.at[slot])

        def row_body(j, carry2):
            def vec_body(k, carry3):
                off = k * 16
                iv = idxsb[slot, j, pl.ds(off, 16)]
                fv = fuseds[slot, pl.ds(j * 128 + off, 16)]
                rel = iv - base
                m = (rel >= 0) & (rel < cw)
                relc = jnp.minimum(jnp.maximum(rel, 0), CHUNK - 1)
                plsc.store_scatter(vals_dst, [relc], fv, mask=m)
                return carry3
            return lax.fori_loop(0, 8, vec_body, carry2)

        return lax.fori_loop(0, ROWS_CH, row_body, carry)

    lax.fori_loop(0, N_CH, chunk_body, 0)

    @pl.when(jnp.logical_not(is_last))
    def _():
        pltpu.sync_copy(vals_dst, out_hbm.at[pl.ds(base, CHUNK)])

    @pl.when(is_last)
    def _():
        pltpu.sync_copy(vals_dst.at[pl.ds(0, CHUNK_LAST)],
                        out_hbm.at[pl.ds(base, CHUNK_LAST)])


_sc_scatter = functools.partial(
    pl.kernel,
    out_type=jax.ShapeDtypeStruct((NUM_NODES,), jnp.float32),
    mesh=plsc.VectorSubcoreMesh(core_axis_name="c", subcore_axis_name="s",
                                num_cores=NC, num_subcores=NS),
    scratch_types=[
        pltpu.VMEM_SHARED((NUM_CAND,), jnp.float32),
        pltpu.VMEM((CHUNK,), jnp.float32),
        pltpu.VMEM((GROWS, 128), jnp.int32),
        pltpu.VMEM((CPT,), jnp.float32),
        pltpu.VMEM((CPT,), jnp.float32),
        pltpu.VMEM((GROWS, 128), jnp.float32),
        pltpu.VMEM((CPT,), jnp.float32),
        pltpu.VMEM((2, ROWS_CH, 128), jnp.int32),
        pltpu.VMEM((2, CAND_CH), jnp.float32),
        pltpu.SemaphoreType.DMA,
    ],
    compiler_params=pltpu.CompilerParams(needs_layout_passes=False),
)(_sc_body)


def kernel(global_emb, local_emb, global_scores, local_scores,
           candidate_indices, W1, b1, W2, b2):
    sigma = _compute_sigma(global_emb, local_emb, W1, b1, W2, b2)
    idx2 = candidate_indices.astype(jnp.int32).reshape(NUM_CAND // 128, 128)
    fused = _sc_scatter(global_scores, idx2, sigma, local_scores)
    return (fused, sigma)


# speedup vs baseline: 2.6442x; 1.3109x over previous
import functools

import jax
import jax.numpy as jnp
from jax import lax
from jax.experimental import pallas as pl
from jax.experimental.pallas import tpu as pltpu
from jax.experimental.pallas import tpu_sc as plsc

NUM_NODES = 1000000
NUM_CAND = 65536
GLOBAL_DIM = 128
LOCAL_DIM = 128
HID = 32

NC = 2
NS = 16
NW = NC * NS

CHUNK = 31256
CHUNK_LAST = NUM_NODES - (NW - 1) * CHUNK

CAND_CH = 4096
N_CH = NUM_CAND // CAND_CH

BR = 2048
GRID = NUM_CAND // BR


def _ffn_body(g_ref, l_ref, w1a_ref, w1b_ref, b1_ref, w2_ref, b2_ref, o_ref):
    dn = (((0,), (1,)), ((), ()))
    ht = lax.dot_general(w1a_ref[...], g_ref[...], dn,
                         preferred_element_type=jnp.float32)
    ht = ht + lax.dot_general(w1b_ref[...], l_ref[...], dn,
                              preferred_element_type=jnp.float32)
    ht = jnp.maximum(ht + b1_ref[...], 0.0)
    st = lax.dot_general(w2_ref[...], ht, (((0,), (0,)), ((), ())),
                         preferred_element_type=jnp.float32)
    o_ref[...] = jax.nn.sigmoid(st + b2_ref[...]).reshape(1, 1, BR)


def _compute_sigma(global_emb, local_emb, W1, b1, W2, b2):
    w1a = W1[:GLOBAL_DIM]
    w1b = W1[GLOBAL_DIM:]
    b1c = b1.reshape(HID, 1)
    b2c = b2.reshape(1, 1)
    out = pl.pallas_call(
        _ffn_body,
        grid=(GRID,),
        in_specs=[
            pl.BlockSpec((BR, GLOBAL_DIM), lambda i: (i, 0)),
            pl.BlockSpec((BR, LOCAL_DIM), lambda i: (i, 0)),
            pl.BlockSpec((GLOBAL_DIM, HID), lambda i: (0, 0)),
            pl.BlockSpec((LOCAL_DIM, HID), lambda i: (0, 0)),
            pl.BlockSpec((HID, 1), lambda i: (0, 0)),
            pl.BlockSpec((HID, 1), lambda i: (0, 0)),
            pl.BlockSpec((1, 1), lambda i: (0, 0)),
        ],
        out_specs=pl.BlockSpec((1, 1, BR), lambda i: (i, 0, 0)),
        out_shape=jax.ShapeDtypeStruct((GRID, 1, BR), jnp.float32),
    )(global_emb, local_emb, w1a, w1b, b1c, W2, b2c)
    return out.reshape(NUM_CAND)


CPT = NUM_CAND // NS
GROWS = CPT // 128
ROWS_CH = CAND_CH // 128


def _sc_body(gs_hbm, idx2_hbm, sig_hbm, loc_hbm, out_hbm,
             fused_sh, vals_dst, idxg, sigb, locb, gbuf, fusedb,
             idxsb, fuseds, gsem, vsem, isem0, isem1, fsem0, fsem1):
    cid = lax.axis_index("c")
    sid = lax.axis_index("s")
    wid = sid * NC + cid
    base = wid * CHUNK
    is_last = wid == NW - 1
    cw = jnp.where(is_last, CHUNK_LAST, CHUNK).astype(jnp.int32)
    isems = (isem0, isem1)
    fsems = (fsem0, fsem1)

    @pl.when(jnp.logical_not(is_last))
    def _():
        pltpu.async_copy(gs_hbm.at[pl.ds(base, CHUNK)], vals_dst, vsem)

    @pl.when(is_last)
    def _():
        pltpu.async_copy(gs_hbm.at[pl.ds(base, CHUNK_LAST)],
                         vals_dst.at[pl.ds(0, CHUNK_LAST)], vsem)

    abase = sid * CPT
    rbase = sid * GROWS
    pltpu.sync_copy(idx2_hbm.at[pl.ds(rbase, GROWS)], idxg)

    @pl.loop(0, GROWS)
    def _(j):
        pltpu.async_copy(gs_hbm.at[idxg.at[j]], gbuf.at[j], gsem)

    pltpu.sync_copy(sig_hbm.at[pl.ds(abase, CPT)], sigb)
    pltpu.sync_copy(loc_hbm.at[pl.ds(abase, CPT)], locb)

    @pl.loop(0, GROWS)
    def _(j):
        pltpu.make_async_copy(gs_hbm.at[idxg.at[j]], gbuf.at[j], gsem).wait()

    def fuse_row(j, carry):
        for k in range(8):
            off = k * 16
            sv = sigb[pl.ds(j * 128 + off, 16)]
            lv = locb[pl.ds(j * 128 + off, 16)]
            gv = gbuf[j, pl.ds(off, 16)]
            fusedb[pl.ds(j * 128 + off, 16)] = sv * gv + (1.0 - sv) * lv
        return carry

    lax.fori_loop(0, GROWS, fuse_row, 0)
    pltpu.sync_copy(fusedb, fused_sh.at[pl.ds(abase, CPT)])

    pltpu.async_copy(idx2_hbm.at[pl.ds(0, ROWS_CH)], idxsb.at[0], isems[0])

    plsc.subcore_barrier()

    pltpu.async_copy(fused_sh.at[pl.ds(0, CAND_CH)], fuseds.at[0], fsems[0])

    @pl.when(jnp.logical_not(is_last))
    def _():
        pltpu.make_async_copy(gs_hbm.at[pl.ds(base, CHUNK)], vals_dst,
                              vsem).wait()

    @pl.when(is_last)
    def _():
        pltpu.make_async_copy(gs_hbm.at[pl.ds(base, CHUNK_LAST)],
                              vals_dst.at[pl.ds(0, CHUNK_LAST)], vsem).wait()

    for c in range(N_CH):
        s = c & 1
        if c + 1 < N_CH:
            ns = 1 - s
            pltpu.async_copy(idx2_hbm.at[pl.ds((c + 1) * ROWS_CH, ROWS_CH)],
                             idxsb.at[ns], isems[ns])
            pltpu.async_copy(fused_sh.at[pl.ds((c + 1) * CAND_CH, CAND_CH)],
                             fuseds.at[ns], fsems[ns])
        pltpu.make_async_copy(idx2_hbm.at[pl.ds(c * ROWS_CH, ROWS_CH)],
                              idxsb.at[s], isems[s]).wait()
        pltpu.make_async_copy(fused_sh.at[pl.ds(c * CAND_CH, CAND_CH)],
                              fuseds.at[s], fsems[s]).wait()

        def row_body(j, carry, s=s):
            for k in range(8):
                off = k * 16
                iv = idxsb[s, j, pl.ds(off, 16)]
                fv = fuseds[s, pl.ds(j * 128 + off, 16)]
                rel = iv - base
                m = (rel >= 0) & (rel < cw)
                relc = jnp.minimum(jnp.maximum(rel, 0), CHUNK - 1)
                plsc.store_scatter(vals_dst, [relc], fv, mask=m)
            return carry

        lax.fori_loop(0, ROWS_CH, row_body, 0)

    @pl.when(jnp.logical_not(is_last))
    def _():
        pltpu.sync_copy(vals_dst, out_hbm.at[pl.ds(base, CHUNK)])

    @pl.when(is_last)
    def _():
        pltpu.sync_copy(vals_dst.at[pl.ds(0, CHUNK_LAST)],
                        out_hbm.at[pl.ds(base, CHUNK_LAST)])


_sc_scatter = functools.partial(
    pl.kernel,
    out_type=jax.ShapeDtypeStruct((NUM_NODES,), jnp.float32),
    mesh=plsc.VectorSubcoreMesh(core_axis_name="c", subcore_axis_name="s",
                                num_cores=NC, num_subcores=NS),
    scratch_types=[
        pltpu.VMEM_SHARED((NUM_CAND,), jnp.float32),
        pltpu.VMEM((CHUNK,), jnp.float32),
        pltpu.VMEM((GROWS, 128), jnp.int32),
        pltpu.VMEM((CPT,), jnp.float32),
        pltpu.VMEM((CPT,), jnp.float32),
        pltpu.VMEM((GROWS, 128), jnp.float32),
        pltpu.VMEM((CPT,), jnp.float32),
        pltpu.VMEM((2, ROWS_CH, 128), jnp.int32),
        pltpu.VMEM((2, CAND_CH), jnp.float32),
        pltpu.SemaphoreType.DMA,
        pltpu.SemaphoreType.DMA,
        pltpu.SemaphoreType.DMA,
        pltpu.SemaphoreType.DMA,
        pltpu.SemaphoreType.DMA,
        pltpu.SemaphoreType.DMA,
    ],
    compiler_params=pltpu.CompilerParams(needs_layout_passes=False),
)(_sc_body)


def kernel(global_emb, local_emb, global_scores, local_scores,
           candidate_indices, W1, b1, W2, b2):
    sigma = _compute_sigma(global_emb, local_emb, W1, b1, W2, b2)
    idx2 = candidate_indices.astype(jnp.int32).reshape(NUM_CAND // 128, 128)
    fused = _sc_scatter(global_scores, idx2, sigma, local_scores)
    return (fused, sigma)


# speedup vs baseline: 3.6401x; 1.3766x over previous
import functools

import jax
import jax.numpy as jnp
from jax import lax
from jax.experimental import pallas as pl
from jax.experimental.pallas import tpu as pltpu
from jax.experimental.pallas import tpu_sc as plsc

NUM_NODES = 1000000
NUM_CAND = 65536
GLOBAL_DIM = 128
LOCAL_DIM = 128
HID = 32

NC = 2
NS = 16
NW = NC * NS

CHUNK = 31256
CHUNK_LAST = NUM_NODES - (NW - 1) * CHUNK

CAND_CH = 4096
N_CH = NUM_CAND // CAND_CH

BR = 4096
GRID = NUM_CAND // BR


def _ffn_body(g_ref, l_ref, w1a_ref, w1b_ref, b1_ref, w2_ref, b2_ref, o_ref):
    dn = (((0,), (1,)), ((), ()))
    ht = lax.dot_general(w1a_ref[...], g_ref[...], dn,
                         preferred_element_type=jnp.float32)
    ht = ht + lax.dot_general(w1b_ref[...], l_ref[...], dn,
                              preferred_element_type=jnp.float32)
    ht = jnp.maximum(ht + b1_ref[...], 0.0)
    st = lax.dot_general(w2_ref[...], ht, (((0,), (0,)), ((), ())),
                         preferred_element_type=jnp.float32)
    o_ref[...] = jax.nn.sigmoid(st + b2_ref[...]).reshape(1, 1, BR)


def _compute_sigma(global_emb, local_emb, W1, b1, W2, b2):
    w1a = W1[:GLOBAL_DIM]
    w1b = W1[GLOBAL_DIM:]
    b1c = b1.reshape(HID, 1)
    b2c = b2.reshape(1, 1)
    out = pl.pallas_call(
        _ffn_body,
        grid=(GRID,),
        in_specs=[
            pl.BlockSpec((BR, GLOBAL_DIM), lambda i: (i, 0)),
            pl.BlockSpec((BR, LOCAL_DIM), lambda i: (i, 0)),
            pl.BlockSpec((GLOBAL_DIM, HID), lambda i: (0, 0)),
            pl.BlockSpec((LOCAL_DIM, HID), lambda i: (0, 0)),
            pl.BlockSpec((HID, 1), lambda i: (0, 0)),
            pl.BlockSpec((HID, 1), lambda i: (0, 0)),
            pl.BlockSpec((1, 1), lambda i: (0, 0)),
        ],
        out_specs=pl.BlockSpec((1, 1, BR), lambda i: (i, 0, 0)),
        out_shape=jax.ShapeDtypeStruct((GRID, 1, BR), jnp.float32),
    )(global_emb, local_emb, w1a, w1b, b1c, W2, b2c)
    return out.reshape(NUM_CAND)


CPT = NUM_CAND // NS
GROWS = CPT // 128
ROWS_CH = CAND_CH // 128


def _sc_body(gs_hbm, idx_hbm, sig_hbm, loc_hbm, out_hbm,
             fused_sh, vals_dst, idxg, sigb, locb, gbuf, fusedb,
             idxsb, fuseds, gsem, vsem, isem0, isem1, fsem0, fsem1):
    cid = lax.axis_index("c")
    sid = lax.axis_index("s")
    wid = sid * NC + cid
    base = wid * CHUNK
    is_last = wid == NW - 1
    cw = jnp.where(is_last, CHUNK_LAST, CHUNK).astype(jnp.int32)
    cwu = cw.astype(jnp.uint32)
    isems = (isem0, isem1)
    fsems = (fsem0, fsem1)

    @pl.when(jnp.logical_not(is_last))
    def _():
        pltpu.async_copy(gs_hbm.at[pl.ds(base, CHUNK)], vals_dst, vsem)

    @pl.when(is_last)
    def _():
        pltpu.async_copy(gs_hbm.at[pl.ds(base, CHUNK_LAST)],
                         vals_dst.at[pl.ds(0, CHUNK_LAST)], vsem)

    abase = sid * CPT
    pltpu.sync_copy(idx_hbm.at[pl.ds(abase, CPT)], idxg)

    @pl.loop(0, GROWS)
    def _(j):
        pltpu.async_copy(gs_hbm.at[idxg.at[pl.ds(j * 128, 128)]],
                         gbuf.at[j], gsem)

    pltpu.sync_copy(sig_hbm.at[pl.ds(abase, CPT)], sigb)
    pltpu.sync_copy(loc_hbm.at[pl.ds(abase, CPT)], locb)

    @pl.loop(0, GROWS)
    def _(j):
        pltpu.make_async_copy(gs_hbm.at[idxg.at[pl.ds(j * 128, 128)]],
                              gbuf.at[j], gsem).wait()

    def fuse_row(j, carry):
        svs = [sigb[pl.ds(j * 128 + k * 16, 16)] for k in range(8)]
        lvs = [locb[pl.ds(j * 128 + k * 16, 16)] for k in range(8)]
        gvs = [gbuf[j, pl.ds(k * 16, 16)] for k in range(8)]
        for k in range(8):
            fusedb[pl.ds(j * 128 + k * 16, 16)] = (
                svs[k] * gvs[k] + (1.0 - svs[k]) * lvs[k])
        return carry

    lax.fori_loop(0, GROWS, fuse_row, 0)
    pltpu.sync_copy(fusedb, fused_sh.at[pl.ds(abase, CPT)])

    pltpu.async_copy(idx_hbm.at[pl.ds(0, CAND_CH)], idxsb.at[0], isems[0])

    plsc.subcore_barrier()

    pltpu.async_copy(fused_sh.at[pl.ds(0, CAND_CH)], fuseds.at[0], fsems[0])

    @pl.when(jnp.logical_not(is_last))
    def _():
        pltpu.make_async_copy(gs_hbm.at[pl.ds(base, CHUNK)], vals_dst,
                              vsem).wait()

    @pl.when(is_last)
    def _():
        pltpu.make_async_copy(gs_hbm.at[pl.ds(base, CHUNK_LAST)],
                              vals_dst.at[pl.ds(0, CHUNK_LAST)], vsem).wait()

    for c in range(N_CH):
        s = c & 1
        if c + 1 < N_CH:
            ns = 1 - s
            pltpu.async_copy(idx_hbm.at[pl.ds((c + 1) * CAND_CH, CAND_CH)],
                             idxsb.at[ns], isems[ns])
            pltpu.async_copy(fused_sh.at[pl.ds((c + 1) * CAND_CH, CAND_CH)],
                             fuseds.at[ns], fsems[ns])
        pltpu.make_async_copy(idx_hbm.at[pl.ds(c * CAND_CH, CAND_CH)],
                              idxsb.at[s], isems[s]).wait()
        pltpu.make_async_copy(fused_sh.at[pl.ds(c * CAND_CH, CAND_CH)],
                              fuseds.at[s], fsems[s]).wait()

        def row_body(j, carry, s=s):
            ivs = [idxsb[s, pl.ds(j * 128 + k * 16, 16)] for k in range(8)]
            fvs = [fuseds[s, pl.ds(j * 128 + k * 16, 16)] for k in range(8)]
            rels = [plsc.bitcast(iv - base, jnp.uint32) for iv in ivs]
            ms = [r < cwu for r in rels]
            relcs = [plsc.bitcast(jnp.minimum(r, jnp.uint32(CHUNK - 1)),
                                  jnp.int32) for r in rels]
            for k in range(8):
                plsc.store_scatter(vals_dst, [relcs[k]], fvs[k], mask=ms[k])
            return carry

        lax.fori_loop(0, ROWS_CH, row_body, 0)

    @pl.when(jnp.logical_not(is_last))
    def _():
        pltpu.sync_copy(vals_dst, out_hbm.at[pl.ds(base, CHUNK)])

    @pl.when(is_last)
    def _():
        pltpu.sync_copy(vals_dst.at[pl.ds(0, CHUNK_LAST)],
                        out_hbm.at[pl.ds(base, CHUNK_LAST)])


_sc_scatter = functools.partial(
    pl.kernel,
    out_type=jax.ShapeDtypeStruct((NUM_NODES,), jnp.float32),
    mesh=plsc.VectorSubcoreMesh(core_axis_name="c", subcore_axis_name="s",
                                num_cores=NC, num_subcores=NS),
    scratch_types=[
        pltpu.VMEM_SHARED((NUM_CAND,), jnp.float32),
        pltpu.VMEM((CHUNK,), jnp.float32),
        pltpu.VMEM((CPT,), jnp.int32),
        pltpu.VMEM((CPT,), jnp.float32),
        pltpu.VMEM((CPT,), jnp.float32),
        pltpu.VMEM((GROWS, 128), jnp.float32),
        pltpu.VMEM((CPT,), jnp.float32),
        pltpu.VMEM((2, CAND_CH), jnp.int32),
        pltpu.VMEM((2, CAND_CH), jnp.float32),
        pltpu.SemaphoreType.DMA,
        pltpu.SemaphoreType.DMA,
        pltpu.SemaphoreType.DMA,
        pltpu.SemaphoreType.DMA,
        pltpu.SemaphoreType.DMA,
        pltpu.SemaphoreType.DMA,
    ],
    compiler_params=pltpu.CompilerParams(needs_layout_passes=False),
)(_sc_body)


def kernel(global_emb, local_emb, global_scores, local_scores,
           candidate_indices, W1, b1, W2, b2):
    sigma = _compute_sigma(global_emb, local_emb, W1, b1, W2, b2)
    idx = candidate_indices.astype(jnp.int32)
    fused = _sc_scatter(global_scores, idx, sigma, local_scores)
    return (fused, sigma)


# speedup vs baseline: 3.9560x; 1.0868x over previous
import functools

import jax
import jax.numpy as jnp
from jax import lax
from jax.experimental import pallas as pl
from jax.experimental.pallas import tpu as pltpu
from jax.experimental.pallas import tpu_sc as plsc

NUM_NODES = 1000000
NUM_CAND = 65536
GLOBAL_DIM = 128
LOCAL_DIM = 128
HID = 32

NC = 2
NS = 16
NW = NC * NS

CHUNK = 31256
CHUNK_LAST = NUM_NODES - (NW - 1) * CHUNK

CAND_CH = 4096
N_CH = NUM_CAND // CAND_CH

BR = 8192
GRID = NUM_CAND // BR


def _ffn_body(g_ref, l_ref, w1a_ref, w1b_ref, b1_ref, w2_ref, b2_ref, o_ref):
    dn = (((0,), (1,)), ((), ()))
    ht = lax.dot_general(w1a_ref[...], g_ref[...], dn,
                         preferred_element_type=jnp.float32)
    ht = ht + lax.dot_general(w1b_ref[...], l_ref[...], dn,
                              preferred_element_type=jnp.float32)
    ht = jnp.maximum(ht + b1_ref[...], 0.0)
    st = lax.dot_general(w2_ref[...], ht, (((0,), (0,)), ((), ())),
                         preferred_element_type=jnp.float32)
    o_ref[...] = jax.nn.sigmoid(st + b2_ref[...]).reshape(1, 1, BR)


def _compute_sigma(global_emb, local_emb, W1, b1, W2, b2):
    w1a = W1[:GLOBAL_DIM]
    w1b = W1[GLOBAL_DIM:]
    b1c = b1.reshape(HID, 1)
    b2c = b2.reshape(1, 1)
    out = pl.pallas_call(
        _ffn_body,
        grid=(GRID,),
        in_specs=[
            pl.BlockSpec((BR, GLOBAL_DIM), lambda i: (i, 0)),
            pl.BlockSpec((BR, LOCAL_DIM), lambda i: (i, 0)),
            pl.BlockSpec((GLOBAL_DIM, HID), lambda i: (0, 0)),
            pl.BlockSpec((LOCAL_DIM, HID), lambda i: (0, 0)),
            pl.BlockSpec((HID, 1), lambda i: (0, 0)),
            pl.BlockSpec((HID, 1), lambda i: (0, 0)),
            pl.BlockSpec((1, 1), lambda i: (0, 0)),
        ],
        out_specs=pl.BlockSpec((1, 1, BR), lambda i: (i, 0, 0)),
        out_shape=jax.ShapeDtypeStruct((GRID, 1, BR), jnp.float32),
    )(global_emb, local_emb, w1a, w1b, b1c, W2, b2c)
    return out.reshape(NUM_CAND)


CPT = NUM_CAND // NS
GROWS = CPT // 128
ROWS_CH = CAND_CH // 128

GPT = NUM_CAND // NW
GPROWS = GPT // 128


def _sc_gather_body(gs_hbm, idx_hbm, g_hbm, idxg, gbuf, gsem):
    cid = lax.axis_index("c")
    sid = lax.axis_index("s")
    wid = sid * NC + cid
    gbase = wid * GPT
    pltpu.sync_copy(idx_hbm.at[pl.ds(gbase, GPT)], idxg)

    @pl.loop(0, GPROWS)
    def _(j):
        pltpu.async_copy(gs_hbm.at[idxg.at[pl.ds(j * 128, 128)]],
                         gbuf.at[j], gsem)

    @pl.loop(0, GPROWS)
    def _(j):
        pltpu.make_async_copy(gs_hbm.at[idxg.at[pl.ds(j * 128, 128)]],
                              gbuf.at[j], gsem).wait()

    pltpu.sync_copy(gbuf, g_hbm.at[pl.ds(wid * GPROWS, GPROWS)])


_sc_gather = functools.partial(
    pl.kernel,
    out_type=jax.ShapeDtypeStruct((NUM_CAND // 128, 128), jnp.float32),
    mesh=plsc.VectorSubcoreMesh(core_axis_name="c", subcore_axis_name="s",
                                num_cores=NC, num_subcores=NS),
    scratch_types=[
        pltpu.VMEM((GPT,), jnp.int32),
        pltpu.VMEM((GPROWS, 128), jnp.float32),
        pltpu.SemaphoreType.DMA,
    ],
    compiler_params=pltpu.CompilerParams(needs_layout_passes=False),
)(_sc_gather_body)


def _sc_body(gs_hbm, idx_hbm, sig_hbm, loc_hbm, g_hbm, out_hbm,
             fused_sh, vals_dst, sigb, locb, gbuf, fusedb,
             idxsb, fuseds, gsem, vsem, isem0, isem1, fsem0, fsem1):
    cid = lax.axis_index("c")
    sid = lax.axis_index("s")
    wid = sid * NC + cid
    base = wid * CHUNK
    is_last = wid == NW - 1
    cw = jnp.where(is_last, CHUNK_LAST, CHUNK).astype(jnp.int32)
    cwu = cw.astype(jnp.uint32)
    isems = (isem0, isem1)
    fsems = (fsem0, fsem1)

    @pl.when(jnp.logical_not(is_last))
    def _():
        pltpu.async_copy(gs_hbm.at[pl.ds(base, CHUNK)], vals_dst, vsem)

    @pl.when(is_last)
    def _():
        pltpu.async_copy(gs_hbm.at[pl.ds(base, CHUNK_LAST)],
                         vals_dst.at[pl.ds(0, CHUNK_LAST)], vsem)

    abase = sid * CPT
    pltpu.async_copy(g_hbm.at[pl.ds(sid * GROWS, GROWS)], gbuf, gsem)
    pltpu.sync_copy(sig_hbm.at[pl.ds(abase, CPT)], sigb)
    pltpu.sync_copy(loc_hbm.at[pl.ds(abase, CPT)], locb)
    pltpu.make_async_copy(g_hbm.at[pl.ds(sid * GROWS, GROWS)], gbuf,
                          gsem).wait()

    def fuse_row(j, carry):
        svs = [sigb[pl.ds(j * 128 + k * 16, 16)] for k in range(8)]
        lvs = [locb[pl.ds(j * 128 + k * 16, 16)] for k in range(8)]
        gvs = [gbuf[j, pl.ds(k * 16, 16)] for k in range(8)]
        for k in range(8):
            fusedb[pl.ds(j * 128 + k * 16, 16)] = (
                svs[k] * gvs[k] + (1.0 - svs[k]) * lvs[k])
        return carry

    lax.fori_loop(0, GROWS, fuse_row, 0)
    pltpu.sync_copy(fusedb, fused_sh.at[pl.ds(abase, CPT)])

    pltpu.async_copy(idx_hbm.at[pl.ds(0, CAND_CH)], idxsb.at[0], isems[0])

    plsc.subcore_barrier()

    pltpu.async_copy(fused_sh.at[pl.ds(0, CAND_CH)], fuseds.at[0], fsems[0])

    @pl.when(jnp.logical_not(is_last))
    def _():
        pltpu.make_async_copy(gs_hbm.at[pl.ds(base, CHUNK)], vals_dst,
                              vsem).wait()

    @pl.when(is_last)
    def _():
        pltpu.make_async_copy(gs_hbm.at[pl.ds(base, CHUNK_LAST)],
                              vals_dst.at[pl.ds(0, CHUNK_LAST)], vsem).wait()

    for c in range(N_CH):
        s = c & 1
        if c + 1 < N_CH:
            ns = 1 - s
            pltpu.async_copy(idx_hbm.at[pl.ds((c + 1) * CAND_CH, CAND_CH)],
                             idxsb.at[ns], isems[ns])
            pltpu.async_copy(fused_sh.at[pl.ds((c + 1) * CAND_CH, CAND_CH)],
                             fuseds.at[ns], fsems[ns])
        pltpu.make_async_copy(idx_hbm.at[pl.ds(c * CAND_CH, CAND_CH)],
                              idxsb.at[s], isems[s]).wait()
        pltpu.make_async_copy(fused_sh.at[pl.ds(c * CAND_CH, CAND_CH)],
                              fuseds.at[s], fsems[s]).wait()

        def row_body(j, carry, s=s):
            ivs = [idxsb[s, pl.ds(j * 128 + k * 16, 16)] for k in range(8)]
            fvs = [fuseds[s, pl.ds(j * 128 + k * 16, 16)] for k in range(8)]
            rels = [plsc.bitcast(iv - base, jnp.uint32) for iv in ivs]
            ms = [r < cwu for r in rels]
            relcs = [plsc.bitcast(jnp.minimum(r, jnp.uint32(CHUNK - 1)),
                                  jnp.int32) for r in rels]
            for k in range(8):
                plsc.store_scatter(vals_dst, [relcs[k]], fvs[k], mask=ms[k])
            return carry

        lax.fori_loop(0, ROWS_CH, row_body, 0)

    @pl.when(jnp.logical_not(is_last))
    def _():
        pltpu.sync_copy(vals_dst, out_hbm.at[pl.ds(base, CHUNK)])

    @pl.when(is_last)
    def _():
        pltpu.sync_copy(vals_dst.at[pl.ds(0, CHUNK_LAST)],
                        out_hbm.at[pl.ds(base, CHUNK_LAST)])


_sc_scatter = functools.partial(
    pl.kernel,
    out_type=jax.ShapeDtypeStruct((NUM_NODES,), jnp.float32),
    mesh=plsc.VectorSubcoreMesh(core_axis_name="c", subcore_axis_name="s",
                                num_cores=NC, num_subcores=NS),
    scratch_types=[
        pltpu.VMEM_SHARED((NUM_CAND,), jnp.float32),
        pltpu.VMEM((CHUNK,), jnp.float32),
        pltpu.VMEM((CPT,), jnp.float32),
        pltpu.VMEM((CPT,), jnp.float32),
        pltpu.VMEM((GROWS, 128), jnp.float32),
        pltpu.VMEM((CPT,), jnp.float32),
        pltpu.VMEM((2, CAND_CH), jnp.int32),
        pltpu.VMEM((2, CAND_CH), jnp.float32),
        pltpu.SemaphoreType.DMA,
        pltpu.SemaphoreType.DMA,
        pltpu.SemaphoreType.DMA,
        pltpu.SemaphoreType.DMA,
        pltpu.SemaphoreType.DMA,
        pltpu.SemaphoreType.DMA,
    ],
    compiler_params=pltpu.CompilerParams(needs_layout_passes=False),
)(_sc_body)


def kernel(global_emb, local_emb, global_scores, local_scores,
           candidate_indices, W1, b1, W2, b2):
    idx = candidate_indices.astype(jnp.int32)
    g2d = _sc_gather(global_scores, idx)
    sigma = _compute_sigma(global_emb, local_emb, W1, b1, W2, b2)
    fused = _sc_scatter(global_scores, idx, sigma, local_scores, g2d)
    return (fused, sigma)
